# baseline (speedup 1.0000x reference)
import jax
import jax.numpy as jnp
from jax import lax
from jax.experimental import pallas as pl
from jax.experimental.pallas import tpu as pltpu
from jax.experimental.pallas import tpu_sc as plsc

N_NODES = 10000
N_INC = 320000
NUM_GRAPHS = 64
D_HID = 128
D_OUT = 10

NC = 2
NS = 16
NT = NC * NS
PER_TILE = 10240
PADDED = NT * PER_TILE
CH = 128
NCHUNKS = PADDED // CH
NCH0 = 160
SEC = 8
NBUF = 2
PD = 1
CH_C = 128
NCH_C = PER_TILE // CH_C
NR = 10240
ZROWS = NR // NS


def _emit_chunk_loop(x_hbm, src_hbm, dst_hbm, y_sh,
                     src_v, dst_v, rows_v, semg, sems, semi, base, nch):
    nsec = nch // SEC
    pltpu.sync_copy(src_hbm.at[pl.ds(base, SEC)], src_v.at[0])
    pltpu.sync_copy(dst_hbm.at[pl.ds(base, SEC)], dst_v.at[0])
    dg = [None] * NBUF
    ds = [None] * NBUF
    di = [None, None]
    for g in range(PD):
        dg[g] = pltpu.async_copy(x_hbm.at[src_v.at[0, g]], rows_v.at[g],
                                 semg.at[g])
    for sec in range(nsec):
        p = sec % 2
        for j in range(SEC):
            g = sec * SEC + j
            if j == PD and sec + 1 < nsec:
                np_ = (sec + 1) % 2
                di[np_] = (
                    pltpu.async_copy(
                        src_hbm.at[pl.ds(base + (sec + 1) * SEC, SEC)],
                        src_v.at[np_], semi.at[0]),
                    pltpu.async_copy(
                        dst_hbm.at[pl.ds(base + (sec + 1) * SEC, SEC)],
                        dst_v.at[np_], semi.at[1]),
                )
            tg = g + PD
            if tg < nch:
                tsec = tg // SEC
                tp = tsec % 2
                tj = tg % SEC
                if tsec != sec and tj == 0:
                    di[tp][0].wait()
                    di[tp][1].wait()
                bb = tg % NBUF
                if tg >= NBUF:
                    ds[bb].wait()
                dg[bb] = pltpu.async_copy(x_hbm.at[src_v.at[tp, tj]],
                                          rows_v.at[bb], semg.at[bb])
            b = g % NBUF
            dg[b].wait()
            ds[b] = pltpu.async_copy(rows_v.at[b], y_sh.at[dst_v.at[p, j]],
                                     sems.at[b], add=True)
    for b in range(NBUF):
        ds[b].wait()


def _scatter_body_impl(x_hbm, src_hbm, dst_hbm, z_hbm, out_hbm,
                       src_v, dst_v, rows_v, y_sh, semg, sems, semi):
    c = lax.axis_index("c")
    s = lax.axis_index("s")

    @pl.when(c == 0)
    def _zero():
        pltpu.sync_copy(z_hbm, y_sh.at[pl.ds(s * ZROWS, ZROWS)])

    plsc.subcore_barrier()

    @pl.when(c == 0)
    def _fast_core():
        _emit_chunk_loop(x_hbm, src_hbm, dst_hbm, y_sh, src_v, dst_v, rows_v,
                         semg, sems, semi, s * NCH0, NCH0)

    plsc.subcore_barrier()

    @pl.when(c == 0)
    def _dump():
        pltpu.sync_copy(y_sh.at[pl.ds(s * ZROWS, ZROWS)],
                        out_hbm.at[pl.ds(s * ZROWS, ZROWS), :])


def _scatter_pass(x, src_idx, dst_idx, zrows):
    mesh = plsc.VectorSubcoreMesh(core_axis_name="c", subcore_axis_name="s",
                                  num_cores=NC, num_subcores=NS)
    f = pl.kernel(
        _scatter_body_impl,
        out_type=jax.ShapeDtypeStruct((NR, D_HID), jnp.float32),
        mesh=mesh,
        scratch_types=[
            pltpu.VMEM((2, SEC, CH), jnp.int32),
            pltpu.VMEM((2, SEC, CH), jnp.int32),
            pltpu.VMEM((NBUF, CH, D_HID), jnp.float32),
            pltpu.VMEM_SHARED((NR, D_HID), jnp.float32),
            pltpu.SemaphoreType.DMA((NBUF,)),
            pltpu.SemaphoreType.DMA((NBUF,)),
            pltpu.SemaphoreType.DMA((2,)),
        ],
    )
    return f(x, src_idx, dst_idx, zrows)


def _count_body(nidx_hbm, eidx_hbm, z_hbm, ones_hbm, outn_hbm, oute_hbm,
                idx_v, ones_v, y_sh, semc):
    c = lax.axis_index("c")
    s = lax.axis_index("s")
    wid = c * NS + s
    pltpu.sync_copy(ones_hbm, ones_v)
    for idx_hbm, out_hbm in ((nidx_hbm, outn_hbm), (eidx_hbm, oute_hbm)):
        pltpu.sync_copy(z_hbm, y_sh.at[pl.ds(s * ZROWS, ZROWS)])
        pltpu.sync_copy(idx_hbm.at[wid], idx_v)
        plsc.subcore_barrier()
        dsc = [None] * 8
        for j in range(NCH_C):
            b = j % 8
            if j >= 8:
                dsc[b].wait()
            dsc[b] = pltpu.async_copy(ones_v, y_sh.at[idx_v.at[j]],
                                      semc.at[b], add=True)
        for b in range(8):
            dsc[b].wait()
        plsc.subcore_barrier()
        pltpu.sync_copy(y_sh.at[pl.ds(s * ZROWS, ZROWS)],
                        out_hbm.at[c, pl.ds(s * ZROWS, ZROWS), :])
        plsc.subcore_barrier()


def _count_pass(nidx, eidx, zrows, ones128):
    mesh = plsc.VectorSubcoreMesh(core_axis_name="c", subcore_axis_name="s",
                                  num_cores=NC, num_subcores=NS)
    f = pl.kernel(
        _count_body,
        out_type=[jax.ShapeDtypeStruct((NC, NR, D_HID), jnp.float32),
                  jax.ShapeDtypeStruct((NC, NR, D_HID), jnp.float32)],
        mesh=mesh,
        scratch_types=[
            pltpu.VMEM((NCH_C, CH_C), jnp.int32),
            pltpu.VMEM((CH_C, D_HID), jnp.float32),
            pltpu.VMEM_SHARED((NR, D_HID), jnp.float32),
            pltpu.SemaphoreType.DMA((8,)),
        ],
    )
    return f(nidx, eidx, zrows, ones128)


_MM_BLK = 1024
_PART_SPEC = pl.BlockSpec((NC, _MM_BLK, D_HID), lambda i: (0, i, 0))
_INV_SPEC = pl.BlockSpec((_MM_BLK, 16), lambda i: (i, 0))
_FULL_OUT = pl.BlockSpec((_MM_BLK, D_HID), lambda i: (i, 0))


def _mm_body(x_ref, w_ref, o_ref):
    o_ref[...] = jnp.dot(x_ref[...], w_ref[...],
                         preferred_element_type=jnp.float32)


def _mm(x, w):
    return pl.pallas_call(
        _mm_body,
        grid=(NR // _MM_BLK,),
        in_specs=[_FULL_OUT, pl.BlockSpec((D_HID, D_HID), lambda i: (0, 0))],
        out_specs=_FULL_OUT,
        out_shape=jax.ShapeDtypeStruct((NR, D_HID), jnp.float32),
    )(x, w)


def _inv_body(c_ref, o_ref):
    cnt = c_ref[0, :, 0:1] + c_ref[1, :, 0:1]
    inv = jnp.where(cnt > 0, 1.0 / jnp.where(cnt > 0, cnt, 1.0), 0.0)
    o_ref[...] = jnp.broadcast_to(inv, o_ref.shape)


def _inv(cnt):
    return pl.pallas_call(
        _inv_body,
        grid=(NR // _MM_BLK,),
        in_specs=[_PART_SPEC],
        out_specs=_INV_SPEC,
        out_shape=jax.ShapeDtypeStruct((NR, 16), jnp.float32),
    )(cnt)


def _scale_body(p_ref, i_ref, o_ref):
    o_ref[...] = p_ref[...] * i_ref[:, 0:1]


def _scale(p, inv):
    return pl.pallas_call(
        _scale_body,
        grid=(NR // _MM_BLK,),
        in_specs=[_FULL_OUT, _INV_SPEC],
        out_specs=_FULL_OUT,
        out_shape=jax.ShapeDtypeStruct((NR, D_HID), jnp.float32),
    )(p, inv)


def _layer_out_body(p_ref, i_ref, b_ref, w_ref, o_ref):
    h = p_ref[...] * i_ref[:, 0:1] + b_ref[...]
    h = jnp.maximum(h, 0.0)
    o_ref[...] = jnp.dot(h, w_ref[...], preferred_element_type=jnp.float32)


def _layer_out(p, inv, b, w):
    return pl.pallas_call(
        _layer_out_body,
        grid=(NR // _MM_BLK,),
        in_specs=[_FULL_OUT, _INV_SPEC,
                  pl.BlockSpec((1, D_HID), lambda i: (0, 0)),
                  pl.BlockSpec((D_HID, D_HID), lambda i: (0, 0))],
        out_specs=_FULL_OUT,
        out_shape=jax.ShapeDtypeStruct((NR, D_HID), jnp.float32),
    )(p, inv, b, w)


_PB = 1000


def _final_body(p_ref, i_ref, b_ref, batch_ref, wl_ref, bl_ref,
                o_ref, acc_s, acc_c):
    i = pl.program_id(0)
    h = p_ref[...] * i_ref[:, 0:1] + b_ref[...]
    h = jnp.maximum(h, 0.0)
    bt = batch_ref[0]
    gids = lax.broadcasted_iota(jnp.int32, (NUM_GRAPHS, 1), 0)
    oh = (bt == gids).astype(jnp.float32)
    ps = jnp.dot(oh, h, preferred_element_type=jnp.float32)
    pc = jnp.dot(oh, jnp.ones_like(h), preferred_element_type=jnp.float32)

    @pl.when(i == 0)
    def _init():
        acc_s[...] = jnp.zeros_like(acc_s)
        acc_c[...] = jnp.zeros_like(acc_c)

    acc_s[...] += ps
    acc_c[...] += pc

    @pl.when(i == pl.num_programs(0) - 1)
    def _fin():
        p = acc_s[...] / jnp.maximum(acc_c[...], 1.0)
        o_ref[...] = jnp.dot(p, wl_ref[...],
                             preferred_element_type=jnp.float32) + bl_ref[...]


def _final(p, inv, b, batch_r, wlp, blp):
    return pl.pallas_call(
        _final_body,
        grid=(N_NODES // _PB,),
        in_specs=[pl.BlockSpec((_PB, D_HID), lambda i: (i, 0)),
                  pl.BlockSpec((_PB, 16), lambda i: (i, 0)),
                  pl.BlockSpec((1, D_HID), lambda i: (0, 0)),
                  pl.BlockSpec((1, 1, _PB), lambda i: (i, 0, 0)),
                  pl.BlockSpec((D_HID, 128), lambda i: (0, 0)),
                  pl.BlockSpec((1, 128), lambda i: (0, 0))],
        out_specs=pl.BlockSpec((NUM_GRAPHS, 128), lambda i: (0, 0)),
        out_shape=jax.ShapeDtypeStruct((NUM_GRAPHS, 128), jnp.float32),
        scratch_shapes=[pltpu.VMEM((NUM_GRAPHS, 128), jnp.float32),
                        pltpu.VMEM((NUM_GRAPHS, 128), jnp.float32)],
    )(p, inv, b, batch_r, wlp, blp)


@jax.jit
def kernel(x, hyperedge_index, batch, W1, b1, W2, b2, Wl, bl):
    nidx = hyperedge_index[0].astype(jnp.int32)
    eidx = hyperedge_index[1].astype(jnp.int32)
    padlen = PADDED - N_INC
    fill = jnp.full((padlen,), N_NODES, jnp.int32)
    nidx_m = jnp.concatenate([nidx, fill]).reshape(NCHUNKS, CH)
    eidx_m = jnp.concatenate([eidx, fill]).reshape(NCHUNKS, CH)
    nidx_c = nidx_m.reshape(NT, NCH_C, CH_C)
    eidx_c = eidx_m.reshape(NT, NCH_C, CH_C)
    xp = jnp.pad(x.astype(jnp.float32), ((0, NR - N_NODES), (0, 0)))

    zrows = jnp.zeros((ZROWS, D_HID), jnp.float32)
    ones128 = jnp.ones((CH_C, D_HID), jnp.float32)

    cntN, cntE = _count_pass(nidx_c, eidx_c, zrows, ones128)
    invN = _inv(cntN)
    invE = _inv(cntE)

    xw1 = _mm(xp, W1)
    m1p = _scatter_pass(xw1, nidx_m, eidx_m, zrows)
    m1 = _scale(m1p, invE)
    o1p = _scatter_pass(m1, eidx_m, nidx_m, zrows)
    xw2 = _layer_out(o1p, invN, b1.reshape(1, D_HID), W2)
    m2p = _scatter_pass(xw2, nidx_m, eidx_m, zrows)
    m2 = _scale(m2p, invE)
    o2p = _scatter_pass(m2, eidx_m, nidx_m, zrows)
    batch_r = batch.astype(jnp.int32).reshape(N_NODES // _PB, 1, _PB)
    wlp = jnp.zeros((D_HID, 128), jnp.float32).at[:, :D_OUT].set(Wl)
    blp = jnp.zeros((1, 128), jnp.float32).at[0, :D_OUT].set(bl)
    logits = _final(o2p, invN, b2.reshape(1, D_HID), batch_r, wlp, blp)
    return logits[:, :D_OUT]


# speedup vs baseline: 1.1329x; 1.1329x over previous
import jax
import jax.numpy as jnp
from jax import lax
from jax.experimental import pallas as pl
from jax.experimental.pallas import tpu as pltpu
from jax.experimental.pallas import tpu_sc as plsc

N_NODES = 10000
N_INC = 320000
NUM_GRAPHS = 64
D_HID = 128
D_OUT = 10

NC = 2
NS = 16
NT = NC * NS
PER_TILE = 10240
PADDED = NT * PER_TILE
CH = 128
NCHUNKS = PADDED // CH
NCH0 = 136
NCH1 = 24
SEC = 8
NBUF = 2
PD = 1
CH_C = 128
NCH_C = PER_TILE // CH_C
NR = 10240
ZROWS = NR // NS


def _emit_chunk_loop(x_hbm, src_hbm, dst_hbm, y_sh,
                     src_v, dst_v, rows_v, semg, sems, semi, base, nch):
    nsec = nch // SEC
    pltpu.sync_copy(src_hbm.at[pl.ds(base, SEC)], src_v.at[0])
    pltpu.sync_copy(dst_hbm.at[pl.ds(base, SEC)], dst_v.at[0])
    dg = [None] * NBUF
    ds = [None] * NBUF
    di = [None, None]
    for g in range(PD):
        dg[g] = pltpu.async_copy(x_hbm.at[src_v.at[0, g]], rows_v.at[g],
                                 semg.at[g])
    for sec in range(nsec):
        p = sec % 2
        for j in range(SEC):
            g = sec * SEC + j
            if j == PD and sec + 1 < nsec:
                np_ = (sec + 1) % 2
                di[np_] = (
                    pltpu.async_copy(
                        src_hbm.at[pl.ds(base + (sec + 1) * SEC, SEC)],
                        src_v.at[np_], semi.at[0]),
                    pltpu.async_copy(
                        dst_hbm.at[pl.ds(base + (sec + 1) * SEC, SEC)],
                        dst_v.at[np_], semi.at[1]),
                )
            tg = g + PD
            if tg < nch:
                tsec = tg // SEC
                tp = tsec % 2
                tj = tg % SEC
                if tsec != sec and tj == 0:
                    di[tp][0].wait()
                    di[tp][1].wait()
                bb = tg % NBUF
                if tg >= NBUF:
                    ds[bb].wait()
                dg[bb] = pltpu.async_copy(x_hbm.at[src_v.at[tp, tj]],
                                          rows_v.at[bb], semg.at[bb])
            b = g % NBUF
            dg[b].wait()
            ds[b] = pltpu.async_copy(rows_v.at[b], y_sh.at[dst_v.at[p, j]],
                                     sems.at[b], add=True)
    for b in range(NBUF):
        ds[b].wait()


def _scatter_body_impl(x_hbm, src_hbm, dst_hbm, z_hbm, out_hbm,
                       src_v, dst_v, rows_v, y_sh, semg, sems, semi):
    c = lax.axis_index("c")
    s = lax.axis_index("s")
    pltpu.sync_copy(z_hbm, y_sh.at[pl.ds(s * ZROWS, ZROWS)])
    plsc.subcore_barrier()

    @pl.when(c == 0)
    def _fast_core():
        _emit_chunk_loop(x_hbm, src_hbm, dst_hbm, y_sh, src_v, dst_v, rows_v,
                         semg, sems, semi, s * NCH0, NCH0)

    @pl.when(c == 1)
    def _slow_core():
        _emit_chunk_loop(x_hbm, src_hbm, dst_hbm, y_sh, src_v, dst_v, rows_v,
                         semg, sems, semi, NS * NCH0 + s * NCH1, NCH1)

    plsc.subcore_barrier()
    pltpu.sync_copy(y_sh.at[pl.ds(s * ZROWS, ZROWS)],
                    out_hbm.at[c, pl.ds(s * ZROWS, ZROWS), :])


def _scatter_pass(x, src_idx, dst_idx, zrows):
    mesh = plsc.VectorSubcoreMesh(core_axis_name="c", subcore_axis_name="s",
                                  num_cores=NC, num_subcores=NS)
    f = pl.kernel(
        _scatter_body_impl,
        out_type=jax.ShapeDtypeStruct((NC, NR, D_HID), jnp.float32),
        mesh=mesh,
        scratch_types=[
            pltpu.VMEM((2, SEC, CH), jnp.int32),
            pltpu.VMEM((2, SEC, CH), jnp.int32),
            pltpu.VMEM((NBUF, CH, D_HID), jnp.float32),
            pltpu.VMEM_SHARED((NR, D_HID), jnp.float32),
            pltpu.SemaphoreType.DMA((NBUF,)),
            pltpu.SemaphoreType.DMA((NBUF,)),
            pltpu.SemaphoreType.DMA((2,)),
        ],
    )
    return f(x, src_idx, dst_idx, zrows)


def _count_body(nidx_hbm, eidx_hbm, z_hbm, ones_hbm, outn_hbm, oute_hbm,
                idx_v, ones_v, y_sh, semc):
    c = lax.axis_index("c")
    s = lax.axis_index("s")
    wid = c * NS + s
    pltpu.sync_copy(ones_hbm, ones_v)
    for idx_hbm, out_hbm in ((nidx_hbm, outn_hbm), (eidx_hbm, oute_hbm)):
        pltpu.sync_copy(z_hbm, y_sh.at[pl.ds(s * ZROWS, ZROWS)])
        pltpu.sync_copy(idx_hbm.at[wid], idx_v)
        plsc.subcore_barrier()
        dsc = [None] * 8
        for j in range(NCH_C):
            b = j % 8
            if j >= 8:
                dsc[b].wait()
            dsc[b] = pltpu.async_copy(ones_v, y_sh.at[idx_v.at[j]],
                                      semc.at[b], add=True)
        for b in range(8):
            dsc[b].wait()
        plsc.subcore_barrier()
        pltpu.sync_copy(y_sh.at[pl.ds(s * ZROWS, ZROWS)],
                        out_hbm.at[c, pl.ds(s * ZROWS, ZROWS), :])
        plsc.subcore_barrier()


def _count_pass(nidx, eidx, zrows, ones128):
    mesh = plsc.VectorSubcoreMesh(core_axis_name="c", subcore_axis_name="s",
                                  num_cores=NC, num_subcores=NS)
    f = pl.kernel(
        _count_body,
        out_type=[jax.ShapeDtypeStruct((NC, NR, D_HID), jnp.float32),
                  jax.ShapeDtypeStruct((NC, NR, D_HID), jnp.float32)],
        mesh=mesh,
        scratch_types=[
            pltpu.VMEM((NCH_C, CH_C), jnp.int32),
            pltpu.VMEM((CH_C, D_HID), jnp.float32),
            pltpu.VMEM_SHARED((NR, D_HID), jnp.float32),
            pltpu.SemaphoreType.DMA((8,)),
        ],
    )
    return f(nidx, eidx, zrows, ones128)


_MM_BLK = 1024
_PART_SPEC = pl.BlockSpec((NC, _MM_BLK, D_HID), lambda i: (0, i, 0))
_INV_SPEC = pl.BlockSpec((_MM_BLK, 16), lambda i: (i, 0))
_FULL_OUT = pl.BlockSpec((_MM_BLK, D_HID), lambda i: (i, 0))


def _mm_body(x_ref, w_ref, o_ref):
    o_ref[...] = jnp.dot(x_ref[...], w_ref[...],
                         preferred_element_type=jnp.float32)


def _mm(x, w):
    return pl.pallas_call(
        _mm_body,
        grid=(NR // _MM_BLK,),
        in_specs=[_FULL_OUT, pl.BlockSpec((D_HID, D_HID), lambda i: (0, 0))],
        out_specs=_FULL_OUT,
        out_shape=jax.ShapeDtypeStruct((NR, D_HID), jnp.float32),
    )(x, w)


def _inv_body(c_ref, o_ref):
    cnt = c_ref[0, :, 0:1] + c_ref[1, :, 0:1]
    inv = jnp.where(cnt > 0, 1.0 / jnp.where(cnt > 0, cnt, 1.0), 0.0)
    o_ref[...] = jnp.broadcast_to(inv, o_ref.shape)


def _inv(cnt):
    return pl.pallas_call(
        _inv_body,
        grid=(NR // _MM_BLK,),
        in_specs=[_PART_SPEC],
        out_specs=_INV_SPEC,
        out_shape=jax.ShapeDtypeStruct((NR, 16), jnp.float32),
    )(cnt)


def _scale_body(p_ref, i_ref, o_ref):
    o_ref[...] = (p_ref[0] + p_ref[1]) * i_ref[:, 0:1]


def _scale(p, inv):
    return pl.pallas_call(
        _scale_body,
        grid=(NR // _MM_BLK,),
        in_specs=[_PART_SPEC, _INV_SPEC],
        out_specs=_FULL_OUT,
        out_shape=jax.ShapeDtypeStruct((NR, D_HID), jnp.float32),
    )(p, inv)


def _layer_out_body(p_ref, i_ref, b_ref, w_ref, o_ref):
    h = (p_ref[0] + p_ref[1]) * i_ref[:, 0:1] + b_ref[...]
    h = jnp.maximum(h, 0.0)
    o_ref[...] = jnp.dot(h, w_ref[...], preferred_element_type=jnp.float32)


def _layer_out(p, inv, b, w):
    return pl.pallas_call(
        _layer_out_body,
        grid=(NR // _MM_BLK,),
        in_specs=[_PART_SPEC, _INV_SPEC,
                  pl.BlockSpec((1, D_HID), lambda i: (0, 0)),
                  pl.BlockSpec((D_HID, D_HID), lambda i: (0, 0))],
        out_specs=_FULL_OUT,
        out_shape=jax.ShapeDtypeStruct((NR, D_HID), jnp.float32),
    )(p, inv, b, w)


_PB = 1000


def _final_body(p_ref, i_ref, b_ref, batch_ref, wl_ref, bl_ref,
                o_ref, acc_s, acc_c):
    i = pl.program_id(0)
    h = (p_ref[0] + p_ref[1]) * i_ref[:, 0:1] + b_ref[...]
    h = jnp.maximum(h, 0.0)
    bt = batch_ref[0]
    gids = lax.broadcasted_iota(jnp.int32, (NUM_GRAPHS, 1), 0)
    oh = (bt == gids).astype(jnp.float32)
    ps = jnp.dot(oh, h, preferred_element_type=jnp.float32)
    pc = jnp.dot(oh, jnp.ones_like(h), preferred_element_type=jnp.float32)

    @pl.when(i == 0)
    def _init():
        acc_s[...] = jnp.zeros_like(acc_s)
        acc_c[...] = jnp.zeros_like(acc_c)

    acc_s[...] += ps
    acc_c[...] += pc

    @pl.when(i == pl.num_programs(0) - 1)
    def _fin():
        p = acc_s[...] / jnp.maximum(acc_c[...], 1.0)
        o_ref[...] = jnp.dot(p, wl_ref[...],
                             preferred_element_type=jnp.float32) + bl_ref[...]


def _final(p, inv, b, batch_r, wlp, blp):
    return pl.pallas_call(
        _final_body,
        grid=(N_NODES // _PB,),
        in_specs=[pl.BlockSpec((NC, _PB, D_HID), lambda i: (0, i, 0)),
                  pl.BlockSpec((_PB, 16), lambda i: (i, 0)),
                  pl.BlockSpec((1, D_HID), lambda i: (0, 0)),
                  pl.BlockSpec((1, 1, _PB), lambda i: (i, 0, 0)),
                  pl.BlockSpec((D_HID, 128), lambda i: (0, 0)),
                  pl.BlockSpec((1, 128), lambda i: (0, 0))],
        out_specs=pl.BlockSpec((NUM_GRAPHS, 128), lambda i: (0, 0)),
        out_shape=jax.ShapeDtypeStruct((NUM_GRAPHS, 128), jnp.float32),
        scratch_shapes=[pltpu.VMEM((NUM_GRAPHS, 128), jnp.float32),
                        pltpu.VMEM((NUM_GRAPHS, 128), jnp.float32)],
    )(p, inv, b, batch_r, wlp, blp)


@jax.jit
def kernel(x, hyperedge_index, batch, W1, b1, W2, b2, Wl, bl):
    nidx = hyperedge_index[0].astype(jnp.int32)
    eidx = hyperedge_index[1].astype(jnp.int32)
    padlen = PADDED - N_INC
    fill = jnp.full((padlen,), N_NODES, jnp.int32)
    nidx_m = jnp.concatenate([nidx, fill]).reshape(NCHUNKS, CH)
    eidx_m = jnp.concatenate([eidx, fill]).reshape(NCHUNKS, CH)
    nidx_c = nidx_m.reshape(NT, NCH_C, CH_C)
    eidx_c = eidx_m.reshape(NT, NCH_C, CH_C)
    xp = jnp.pad(x.astype(jnp.float32), ((0, NR - N_NODES), (0, 0)))

    zrows = jnp.zeros((ZROWS, D_HID), jnp.float32)
    ones128 = jnp.ones((CH_C, D_HID), jnp.float32)

    cntN, cntE = _count_pass(nidx_c, eidx_c, zrows, ones128)
    invN = _inv(cntN)
    invE = _inv(cntE)

    xw1 = _mm(xp, W1)
    m1p = _scatter_pass(xw1, nidx_m, eidx_m, zrows)
    m1 = _scale(m1p, invE)
    o1p = _scatter_pass(m1, eidx_m, nidx_m, zrows)
    xw2 = _layer_out(o1p, invN, b1.reshape(1, D_HID), W2)
    m2p = _scatter_pass(xw2, nidx_m, eidx_m, zrows)
    m2 = _scale(m2p, invE)
    o2p = _scatter_pass(m2, eidx_m, nidx_m, zrows)
    batch_r = batch.astype(jnp.int32).reshape(N_NODES // _PB, 1, _PB)
    wlp = jnp.zeros((D_HID, 128), jnp.float32).at[:, :D_OUT].set(Wl)
    blp = jnp.zeros((1, 128), jnp.float32).at[0, :D_OUT].set(bl)
    logits = _final(o2p, invN, b2.reshape(1, D_HID), batch_r, wlp, blp)
    return logits[:, :D_OUT]
